# baseline (speedup 1.0000x reference)
import jax
import jax.numpy as jnp
from jax.experimental import pallas as pl
from jax.experimental.pallas import tpu as pltpu

B, N = 4, 512
RNN, INTER = 128, 256


def _encoder_kernel(f_ref, w1_ref, b1_ref, w2_ref, b2_ref, w3_ref, b3_ref,
                    out_ref, fn_s, x1_s, dsq_s, w23_s, b23_s):
    j = pl.program_id(0)

    mm = lambda a, b: jax.lax.dot_general(
        a, b, (((1,), (0,)), ((), ())), preferred_element_type=jnp.float32)

    def s_apply(x, fn, dsq):
        z = jax.lax.dot_general(
            fn, x, (((0,), (0,)), ((), ())),
            preferred_element_type=jnp.float32)
        return z + x * dsq[:, None]

    @pl.when(j == 0)
    def _():
        w23_s[...] = mm(w2_ref[...], w3_ref[...])
        b23_s[...] = mm(b2_ref[...][None, :], w3_ref[...])

    @pl.when(j > 0)
    def _():
        fn = fn_s[...]
        dsq = dsq_s[0]
        t1 = s_apply(x1_s[...], fn, dsq)
        h3 = mm(t1, w23_s[...]) + b23_s[...]
        out_ref[...] = s_apply(h3, fn, dsq) + b3_ref[...][None, :]

    @pl.when(j < B)
    def _():
        f = f_ref[0]
        deg = jnp.sum(f, axis=0) + 1.0
        dinv = jax.lax.rsqrt(deg)
        dsq = dinv * dinv
        fn = f * dinv[:, None] * dinv[None, :]
        h1 = mm(f, w1_ref[...])
        x1 = s_apply(h1, fn, dsq) + b1_ref[...][None, :]
        fn_s[...] = fn
        x1_s[...] = x1
        dsq_s[0] = dsq


def kernel(flows, W1, b1, W2, b2, W3, b3):
    full = lambda shape: pl.BlockSpec(shape, lambda j: (0,) * len(shape))
    return pl.pallas_call(
        _encoder_kernel,
        grid=(B + 1,),
        in_specs=[
            pl.BlockSpec((1, N, N), lambda j: (jnp.minimum(j, B - 1), 0, 0)),
            full((N, RNN)),
            full((RNN,)),
            full((RNN, INTER)),
            full((INTER,)),
            full((INTER, RNN)),
            full((RNN,)),
        ],
        out_specs=pl.BlockSpec(
            (None, N, RNN), lambda j: (jnp.maximum(j - 1, 0), 0, 0)),
        out_shape=jax.ShapeDtypeStruct((B, N, RNN), jnp.float32),
        scratch_shapes=[
            pltpu.VMEM((N, N), jnp.float32),
            pltpu.VMEM((N, RNN), jnp.float32),
            pltpu.VMEM((1, N), jnp.float32),
            pltpu.VMEM((RNN, RNN), jnp.float32),
            pltpu.VMEM((1, RNN), jnp.float32),
        ],
    )(flows, W1, b1, W2, b2, W3, b3)


# speedup vs baseline: 1.5020x; 1.5020x over previous
import jax
import jax.numpy as jnp
from jax.experimental import pallas as pl
from jax.experimental.pallas import tpu as pltpu

B, N = 4, 512
RNN, INTER = 128, 256


def _encoder_kernel(f_ref, w1_ref, b1_ref, w2_ref, b2_ref, w3_ref, b3_ref,
                    out_ref):
    mm = lambda a, b: jax.lax.dot_general(
        a, b, (((1,), (0,)), ((), ())), preferred_element_type=jnp.float32)

    w23 = mm(w2_ref[...], w3_ref[...])
    b23 = mm(b2_ref[...][None, :], w3_ref[...])

    fs, fnts, dsqs, sapps = [], [], [], []
    for j in range(2):
        f = f_ref[j]
        deg = jnp.sum(f, axis=0) + 1.0
        dinv = jax.lax.rsqrt(deg)
        dsq = dinv * dinv
        fn = f * dinv[:, None] * dinv[None, :]

        def s_apply(x, fn=fn, dsq=dsq):
            z = jax.lax.dot_general(
                fn, x, (((0,), (0,)), ((), ())),
                preferred_element_type=jnp.float32)
            return z + x * dsq[:, None]

        fs.append(f)
        sapps.append(s_apply)

    h1 = [mm(fs[j], w1_ref[...]) for j in range(2)]
    x1 = [sapps[j](h1[j]) + b1_ref[...][None, :] for j in range(2)]
    t1 = [sapps[j](x1[j]) for j in range(2)]
    h3 = [mm(t1[j], w23) + b23 for j in range(2)]
    for j in range(2):
        out_ref[j] = sapps[j](h3[j]) + b3_ref[...][None, :]


def kernel(flows, W1, b1, W2, b2, W3, b3):
    full = lambda shape: pl.BlockSpec(shape, lambda b: (0,) * len(shape))
    return pl.pallas_call(
        _encoder_kernel,
        grid=(B // 2,),
        in_specs=[
            pl.BlockSpec((2, N, N), lambda b: (b, 0, 0)),
            full((N, RNN)),
            full((RNN,)),
            full((RNN, INTER)),
            full((INTER,)),
            full((INTER, RNN)),
            full((RNN,)),
        ],
        out_specs=pl.BlockSpec((2, N, RNN), lambda b: (b, 0, 0)),
        out_shape=jax.ShapeDtypeStruct((B, N, RNN), jnp.float32),
        compiler_params=pltpu.CompilerParams(dimension_semantics=("parallel",)),
    )(flows, W1, b1, W2, b2, W3, b3)
